# baseline (speedup 1.0000x reference)
import jax
import jax.numpy as jnp
from jax.experimental import pallas as pl
from jax.experimental.pallas import tpu as pltpu

B, S, CIN = 4, 8192, 1024
STRIDE = 4
COUT = CIN // STRIDE
ROWS = B * S

RB = 512
GRID = ROWS // RB


def _tc_body(x_ref, o_ref):
    lane = jax.lax.broadcasted_iota(jnp.int32, (RB, 128), 1)
    idx = (lane % 32) * STRIDE
    for g in range(CIN // 128):
        src = x_ref[:, g * 128:(g + 1) * 128]
        gathered = jnp.take_along_axis(src, idx, axis=1)
        o_ref[:, g * 32:(g + 1) * 32] = gathered[:, :32]


_tc = pl.pallas_call(
    _tc_body,
    grid=(GRID,),
    in_specs=[pl.BlockSpec((RB, CIN), lambda i: (i, 0))],
    out_specs=pl.BlockSpec((RB, COUT), lambda i: (i, 0)),
    out_shape=jax.ShapeDtypeStruct((ROWS, COUT), jnp.float32),
)


def kernel(x):
    return _tc(x.reshape(ROWS, CIN)).reshape(B, S, COUT)
